# baseline (speedup 1.0000x reference)
import functools

import jax
import jax.numpy as jnp
from jax import lax
from jax.experimental import pallas as pl
from jax.experimental.pallas import tpu as pltpu
from jax.experimental.pallas import tpu_sc as plsc

_VB = 65536
_VT_BLOCKS = 1
_SC_LANES = 16


def _sc_worker(x_hbm, max_out, idx_out, buf0, buf1, outv, outi, sem0, sem1,
               *, B, VT, CH, CHUNKS, VSC_END):
    w = lax.axis_index("s") * 2 + lax.axis_index("c")

    @pl.when(w < B)
    def _():
        bufs = (buf0, buf1)
        sems = (sem0, sem1)
        off0, len0 = CHUNKS[0]
        h_next = pltpu.async_copy(x_hbm.at[w, 0, pl.ds(off0, len0)],
                                  buf0.at[pl.ds(0, len0)], sem0)
        best_m = jnp.full((_SC_LANES,), -jnp.inf, jnp.float32)
        best_off = jnp.full((_SC_LANES,), VT, jnp.int32)
        for k, (off, ln) in enumerate(CHUNKS):
            h_cur = h_next
            if k + 1 < len(CHUNKS):
                noff, nln = CHUNKS[k + 1]
                nb = (k + 1) % 2
                h_next = pltpu.async_copy(
                    x_hbm.at[w, 0, pl.ds(noff, nln)],
                    bufs[nb].at[pl.ds(0, nln)], sems[nb])
            h_cur.wait()
            buf = bufs[k % 2]
            U = 16
            step = _SC_LANES * U
            ninf = jnp.full((_SC_LANES,), -jnp.inf, jnp.float32)

            def body(j, carry, buf=buf):
                vs = list(carry)
                for u in range(U):
                    vs[u] = jnp.maximum(
                        vs[u], buf[pl.ds(j * step + u * _SC_LANES, _SC_LANES)])
                return tuple(vs)

            vs = lax.fori_loop(0, ln // step, body, (ninf,) * U)
            vm = functools.reduce(jnp.maximum, vs)
            nrem = (ln % step) // _SC_LANES

            def body_r(j, vmc, buf=buf, base=(ln // step) * step):
                return jnp.maximum(vmc, buf[pl.ds(base + j * _SC_LANES, _SC_LANES)])

            if nrem:
                vm = lax.fori_loop(0, nrem, body_r, vm)
            upd = vm > best_m
            best_off = jnp.where(upd, jnp.int32(off), best_off)
            best_m = jnp.maximum(best_m, vm)

        outv[...] = best_m
        outi[...] = best_off
        pltpu.sync_copy(outv, max_out.at[w])
        pltpu.sync_copy(outi, idx_out.at[w])


_SC_CH = 28672


def _merge_sc(scm_ref, scoff_ref, woff_out, *, B, CH, VSC_END):
    big = jnp.int32(2**31 - 1)
    scm = scm_ref[...]
    m = jnp.max(scm, axis=1, keepdims=True)
    woff = jnp.min(jnp.where(scm == m, scoff_ref[...], big), axis=1, keepdims=True)
    woff_out[...] = jnp.minimum(woff, VSC_END - CH)


def _sc_partial(logits, B, VT, VSC_END):
    CH = _SC_CH
    chunks = []
    off = VT
    while off < VSC_END:
        ln = min(CH, VSC_END - off)
        chunks.append((off, ln))
        off += ln
    mesh = plsc.VectorSubcoreMesh(core_axis_name="c", subcore_axis_name="s",
                                  num_cores=2)
    f = pl.kernel(
        functools.partial(_sc_worker, B=B, VT=VT, CH=CH, CHUNKS=tuple(chunks),
                          VSC_END=VSC_END),
        out_type=(
            jax.ShapeDtypeStruct((B, _SC_LANES), jnp.float32),
            jax.ShapeDtypeStruct((B, _SC_LANES), jnp.int32),
        ),
        mesh=mesh,
        scratch_types=[
            pltpu.VMEM((CH,), jnp.float32),
            pltpu.VMEM((CH,), jnp.float32),
            pltpu.VMEM((_SC_LANES,), jnp.float32),
            pltpu.VMEM((_SC_LANES,), jnp.int32),
            pltpu.SemaphoreType.DMA,
            pltpu.SemaphoreType.DMA,
        ],
    )
    return f(logits)


def _pass_a(x_ref, bid_out, max_out, vmax_ref, vbid_ref, *, B, V, NB):
    i = pl.program_id(0)

    @pl.when(i == 0)
    def _init():
        vmax_ref[...] = jnp.full((B, 1), -jnp.inf, jnp.float32)
        vbid_ref[...] = jnp.zeros((B, 1), jnp.int32)

    def _update(bmax):
        better = bmax > vmax_ref[...]
        vbid_ref[...] = jnp.where(better, i, vbid_ref[...])
        vmax_ref[...] = jnp.where(better, bmax, vmax_ref[...])

    @pl.when(i < NB - 1)
    def _full():
        _update(jnp.max(x_ref[...].reshape(B, _VB), axis=1, keepdims=True))

    @pl.when(i == NB - 1)
    def _tail():
        rem = V - (NB - 1) * _VB
        lidx = jax.lax.broadcasted_iota(jnp.int32, (B, _VB), 1)
        x = jnp.where(lidx < rem, x_ref[...].reshape(B, _VB), -jnp.inf)
        _update(jnp.max(x, axis=1, keepdims=True))
        bid_out[...] = vbid_ref[...]
        max_out[...] = vmax_ref[...]


def _pass_b(bid_sref, woff_sref, gi_ref, x_any, bidv_ref, max_ref, lti_ref,
            am_ref, gt_ref, scm_ref, woffv_ref,
            tok_out, lti_out, am_out, gt_out, gi_out, xbuf, tbuf, scbuf, sem,
            *, B, V, S):
    amax = ((V - _VB) // 128) * 128
    tw = 128 + (V % 128 or 128)
    toff = V - tw
    copies = []
    for b in range(B):
        off = pl.multiple_of(jnp.minimum(bid_sref[b] * _VB, amax), 128)
        copies.append(pltpu.make_async_copy(
            x_any.at[b, 0, pl.ds(off, _VB)], xbuf.at[b], sem))
        copies.append(pltpu.make_async_copy(
            x_any.at[b, 0, pl.ds(toff, tw)], tbuf.at[b], sem))
        woff = pl.multiple_of(woff_sref[b], 128)
        copies.append(pltpu.make_async_copy(
            x_any.at[b, 0, pl.ds(woff, _SC_CH)], scbuf.at[b], sem))
    for c in copies:
        c.start()
    for c in copies:
        c.wait()

    big = jnp.int32(2**31 - 1)
    x = xbuf[...]
    base = jnp.minimum(bidv_ref[...] * _VB, amax)
    lidx = jax.lax.broadcasted_iota(jnp.int32, (B, _VB), 1)
    cand = jnp.where(x == max_ref[...], lidx + base, big)
    tc_tok = jnp.min(cand, axis=1, keepdims=True)
    t = tbuf[...]
    tmax = jnp.max(t, axis=1, keepdims=True)
    tidx = jax.lax.broadcasted_iota(jnp.int32, (B, tw), 1) + toff
    cand2 = jnp.where(t == tmax, tidx, big)
    t_tok = jnp.min(cand2, axis=1, keepdims=True)
    sc_m = jnp.max(scm_ref[...], axis=1, keepdims=True)
    xs = scbuf[...]
    sidx = jax.lax.broadcasted_iota(jnp.int32, (B, _SC_CH), 1) + woffv_ref[...]
    sc_i = jnp.min(jnp.where(xs == sc_m, sidx, big), axis=1, keepdims=True)
    v1 = jnp.maximum(max_ref[...], sc_m)
    tok1 = jnp.where(max_ref[...] >= sc_m, tc_tok, sc_i)
    tokens = jnp.where(v1 >= tmax, tok1, t_tok)
    tok_out[...] = tokens
    lti = jnp.minimum(lti_ref[...] + 1, S - 1)
    lti_out[...] = lti
    scol = jax.lax.broadcasted_iota(jnp.int32, (B, S), 1)
    am_out[...] = jnp.where(scol == lti, 1, am_ref[...])
    gi = gi_ref[0]
    gt_out[...] = jnp.where(scol == gi, tokens, gt_ref[...])
    gi_out[0] = jnp.minimum(gi + 1, S - 1)


def kernel(logits, last_token_index, attention_mask, generated_tokens, generated_index):
    B, _, V = logits.shape
    S = generated_tokens.shape[1]
    VT = _VT_BLOCKS * _VB
    VSC_END = (V // 128) * 128
    NB = VT // _VB

    sc_max, sc_off = _sc_partial(logits, B, VT, VSC_END)
    woff = pl.pallas_call(
        functools.partial(_merge_sc, B=B, CH=_SC_CH, VSC_END=VSC_END),
        out_shape=jax.ShapeDtypeStruct((B, 1), jnp.int32),
    )(sc_max, sc_off)

    bid, vmax = pl.pallas_call(
        functools.partial(_pass_a, B=B, V=VT, NB=NB),
        grid=(NB,),
        in_specs=[pl.BlockSpec((B, 1, _VB), lambda i: (0, 0, i))],
        out_specs=[
            pl.BlockSpec((B, 1), lambda i: (0, 0)),
            pl.BlockSpec((B, 1), lambda i: (0, 0)),
        ],
        out_shape=(
            jax.ShapeDtypeStruct((B, 1), jnp.int32),
            jax.ShapeDtypeStruct((B, 1), jnp.float32),
        ),
        scratch_shapes=[
            pltpu.VMEM((B, 1), jnp.float32),
            pltpu.VMEM((B, 1), jnp.int32),
        ],
        compiler_params=pltpu.CompilerParams(
            dimension_semantics=("arbitrary",),
        ),
    )(logits)

    const = lambda i, bid_ref, woff_ref, gi_ref: (0, 0)
    grid_spec = pltpu.PrefetchScalarGridSpec(
        num_scalar_prefetch=3,
        grid=(1,),
        in_specs=[
            pl.BlockSpec(memory_space=pl.ANY),
            pl.BlockSpec((B, 1), const),
            pl.BlockSpec((B, 1), const),
            pl.BlockSpec((B, 1), const),
            pl.BlockSpec((B, S), const),
            pl.BlockSpec((B, S), const),
            pl.BlockSpec((B, _SC_LANES), const),
            pl.BlockSpec((B, 1), const),
        ],
        out_specs=[
            pl.BlockSpec((B, 1), const),
            pl.BlockSpec((B, 1), const),
            pl.BlockSpec((B, S), const),
            pl.BlockSpec((B, S), const),
            pl.BlockSpec(memory_space=pltpu.SMEM),
        ],
        scratch_shapes=[
            pltpu.VMEM((B, _VB), jnp.float32),
            pltpu.VMEM((B, 128 + (V % 128 or 128)), jnp.float32),
            pltpu.VMEM((B, _SC_CH), jnp.float32),
            pltpu.SemaphoreType.DMA,
        ],
    )
    tok, lti, am, gt, gi = pl.pallas_call(
        functools.partial(_pass_b, B=B, V=V, S=S),
        grid_spec=grid_spec,
        out_shape=(
            jax.ShapeDtypeStruct((B, 1), jnp.int32),
            jax.ShapeDtypeStruct((B, 1), jnp.int32),
            jax.ShapeDtypeStruct((B, S), attention_mask.dtype),
            jax.ShapeDtypeStruct((B, S), generated_tokens.dtype),
            jax.ShapeDtypeStruct((1,), jnp.int32),
        ),
        compiler_params=pltpu.CompilerParams(
            dimension_semantics=("arbitrary",),
        ),
    )(bid.reshape(B), woff.reshape(B), generated_index, logits, bid, vmax,
      last_token_index, attention_mask, generated_tokens, sc_max, woff)
    return tok, lti, am, gt, gi


# speedup vs baseline: 1.1655x; 1.1655x over previous
import functools

import jax
import jax.numpy as jnp
from jax import lax
from jax.experimental import pallas as pl
from jax.experimental.pallas import tpu as pltpu
from jax.experimental.pallas import tpu_sc as plsc

_VB = 65536
_VT_BLOCKS = 13
_SC_LANES = 16


def _sc_worker(x_hbm, max_out, idx_out, buf0, buf1, outv, outi, sem0, sem1,
               *, B, VT, CH, CHUNKS, VSC_END):
    w = lax.axis_index("s") * 2 + lax.axis_index("c")

    @pl.when(w < B)
    def _():
        bufs = (buf0, buf1)
        sems = (sem0, sem1)
        off0, len0 = CHUNKS[0]
        h_next = pltpu.async_copy(x_hbm.at[w, 0, pl.ds(off0, len0)],
                                  buf0.at[pl.ds(0, len0)], sem0)
        best_m = jnp.full((_SC_LANES,), -jnp.inf, jnp.float32)
        best_off = jnp.full((_SC_LANES,), VT, jnp.int32)
        for k, (off, ln) in enumerate(CHUNKS):
            h_cur = h_next
            if k + 1 < len(CHUNKS):
                noff, nln = CHUNKS[k + 1]
                nb = (k + 1) % 2
                h_next = pltpu.async_copy(
                    x_hbm.at[w, 0, pl.ds(noff, nln)],
                    bufs[nb].at[pl.ds(0, nln)], sems[nb])
            h_cur.wait()
            buf = bufs[k % 2]
            U = 16
            step = _SC_LANES * U
            ninf = jnp.full((_SC_LANES,), -jnp.inf, jnp.float32)

            def body(j, carry, buf=buf):
                vs = list(carry)
                for u in range(U):
                    vs[u] = jnp.maximum(
                        vs[u], buf[pl.ds(j * step + u * _SC_LANES, _SC_LANES)])
                return tuple(vs)

            vs = lax.fori_loop(0, ln // step, body, (ninf,) * U)
            vm = functools.reduce(jnp.maximum, vs)
            nrem = (ln % step) // _SC_LANES

            def body_r(j, vmc, buf=buf, base=(ln // step) * step):
                return jnp.maximum(vmc, buf[pl.ds(base + j * _SC_LANES, _SC_LANES)])

            if nrem:
                vm = lax.fori_loop(0, nrem, body_r, vm)
            upd = vm > best_m
            best_off = jnp.where(upd, jnp.int32(off), best_off)
            best_m = jnp.maximum(best_m, vm)

        outv[...] = best_m
        outi[...] = best_off
        pltpu.sync_copy(outv, max_out.at[w])
        pltpu.sync_copy(outi, idx_out.at[w])


_SC_CH = 28672


def _merge_sc(scm_ref, scoff_ref, woff_out, *, B, CH, VSC_END):
    big = jnp.int32(2**31 - 1)
    scm = scm_ref[...]
    m = jnp.max(scm, axis=1, keepdims=True)
    woff = jnp.min(jnp.where(scm == m, scoff_ref[...], big), axis=1, keepdims=True)
    woff_out[...] = jnp.minimum(woff, VSC_END - CH)


def _sc_partial(logits, B, VT, VSC_END):
    CH = _SC_CH
    chunks = []
    off = VT
    while off < VSC_END:
        ln = min(CH, VSC_END - off)
        chunks.append((off, ln))
        off += ln
    mesh = plsc.VectorSubcoreMesh(core_axis_name="c", subcore_axis_name="s",
                                  num_cores=2)
    f = pl.kernel(
        functools.partial(_sc_worker, B=B, VT=VT, CH=CH, CHUNKS=tuple(chunks),
                          VSC_END=VSC_END),
        out_type=(
            jax.ShapeDtypeStruct((B, _SC_LANES), jnp.float32),
            jax.ShapeDtypeStruct((B, _SC_LANES), jnp.int32),
        ),
        mesh=mesh,
        scratch_types=[
            pltpu.VMEM((CH,), jnp.float32),
            pltpu.VMEM((CH,), jnp.float32),
            pltpu.VMEM((_SC_LANES,), jnp.float32),
            pltpu.VMEM((_SC_LANES,), jnp.int32),
            pltpu.SemaphoreType.DMA,
            pltpu.SemaphoreType.DMA,
        ],
    )
    return f(logits)


def _pass_a(x_ref, bid_out, max_out, vmax_ref, vbid_ref, *, B, V, NB):
    i = pl.program_id(0)

    @pl.when(i == 0)
    def _init():
        vmax_ref[...] = jnp.full((B, 1), -jnp.inf, jnp.float32)
        vbid_ref[...] = jnp.zeros((B, 1), jnp.int32)

    def _update(bmax):
        better = bmax > vmax_ref[...]
        vbid_ref[...] = jnp.where(better, i, vbid_ref[...])
        vmax_ref[...] = jnp.where(better, bmax, vmax_ref[...])

    @pl.when(i < NB - 1)
    def _full():
        _update(jnp.max(x_ref[...].reshape(B, _VB), axis=1, keepdims=True))

    @pl.when(i == NB - 1)
    def _tail():
        rem = V - (NB - 1) * _VB
        lidx = jax.lax.broadcasted_iota(jnp.int32, (B, _VB), 1)
        x = jnp.where(lidx < rem, x_ref[...].reshape(B, _VB), -jnp.inf)
        _update(jnp.max(x, axis=1, keepdims=True))
        bid_out[...] = vbid_ref[...]
        max_out[...] = vmax_ref[...]


def _pass_b(bid_sref, woff_sref, gi_ref, x_any, bidv_ref, max_ref, lti_ref,
            am_ref, gt_ref, scm_ref, woffv_ref,
            tok_out, lti_out, am_out, gt_out, gi_out, xbuf, tbuf, scbuf, sem,
            *, B, V, S):
    amax = ((V - _VB) // 128) * 128
    tw = 128 + (V % 128 or 128)
    toff = V - tw
    copies = []
    for b in range(B):
        off = pl.multiple_of(jnp.minimum(bid_sref[b] * _VB, amax), 128)
        copies.append(pltpu.make_async_copy(
            x_any.at[b, 0, pl.ds(off, _VB)], xbuf.at[b], sem))
        copies.append(pltpu.make_async_copy(
            x_any.at[b, 0, pl.ds(toff, tw)], tbuf.at[b], sem))
        woff = pl.multiple_of(woff_sref[b], 128)
        copies.append(pltpu.make_async_copy(
            x_any.at[b, 0, pl.ds(woff, _SC_CH)], scbuf.at[b], sem))
    for c in copies:
        c.start()
    for c in copies:
        c.wait()

    big = jnp.int32(2**31 - 1)
    x = xbuf[...]
    base = jnp.minimum(bidv_ref[...] * _VB, amax)
    lidx = jax.lax.broadcasted_iota(jnp.int32, (B, _VB), 1)
    cand = jnp.where(x == max_ref[...], lidx + base, big)
    tc_tok = jnp.min(cand, axis=1, keepdims=True)
    t = tbuf[...]
    tmax = jnp.max(t, axis=1, keepdims=True)
    tidx = jax.lax.broadcasted_iota(jnp.int32, (B, tw), 1) + toff
    cand2 = jnp.where(t == tmax, tidx, big)
    t_tok = jnp.min(cand2, axis=1, keepdims=True)
    sc_m = jnp.max(scm_ref[...], axis=1, keepdims=True)
    xs = scbuf[...]
    sidx = jax.lax.broadcasted_iota(jnp.int32, (B, _SC_CH), 1) + woffv_ref[...]
    sc_i = jnp.min(jnp.where(xs == sc_m, sidx, big), axis=1, keepdims=True)
    v1 = jnp.maximum(max_ref[...], sc_m)
    tok1 = jnp.where(max_ref[...] >= sc_m, tc_tok, sc_i)
    tokens = jnp.where(v1 >= tmax, tok1, t_tok)
    tok_out[...] = tokens
    lti = jnp.minimum(lti_ref[...] + 1, S - 1)
    lti_out[...] = lti
    scol = jax.lax.broadcasted_iota(jnp.int32, (B, S), 1)
    am_out[...] = jnp.where(scol == lti, 1, am_ref[...])
    gi = gi_ref[0]
    gt_out[...] = jnp.where(scol == gi, tokens, gt_ref[...])
    gi_out[0] = jnp.minimum(gi + 1, S - 1)


def kernel(logits, last_token_index, attention_mask, generated_tokens, generated_index):
    B, _, V = logits.shape
    S = generated_tokens.shape[1]
    VT = _VT_BLOCKS * _VB
    VSC_END = (V // 128) * 128
    NB = VT // _VB

    sc_max, sc_off = _sc_partial(logits, B, VT, VSC_END)
    woff = pl.pallas_call(
        functools.partial(_merge_sc, B=B, CH=_SC_CH, VSC_END=VSC_END),
        out_shape=jax.ShapeDtypeStruct((B, 1), jnp.int32),
    )(sc_max, sc_off)

    bid, vmax = pl.pallas_call(
        functools.partial(_pass_a, B=B, V=VT, NB=NB),
        grid=(NB,),
        in_specs=[pl.BlockSpec((B, 1, _VB), lambda i: (0, 0, i))],
        out_specs=[
            pl.BlockSpec((B, 1), lambda i: (0, 0)),
            pl.BlockSpec((B, 1), lambda i: (0, 0)),
        ],
        out_shape=(
            jax.ShapeDtypeStruct((B, 1), jnp.int32),
            jax.ShapeDtypeStruct((B, 1), jnp.float32),
        ),
        scratch_shapes=[
            pltpu.VMEM((B, 1), jnp.float32),
            pltpu.VMEM((B, 1), jnp.int32),
        ],
        compiler_params=pltpu.CompilerParams(
            dimension_semantics=("arbitrary",),
        ),
    )(logits)

    const = lambda i, bid_ref, woff_ref, gi_ref: (0, 0)
    grid_spec = pltpu.PrefetchScalarGridSpec(
        num_scalar_prefetch=3,
        grid=(1,),
        in_specs=[
            pl.BlockSpec(memory_space=pl.ANY),
            pl.BlockSpec((B, 1), const),
            pl.BlockSpec((B, 1), const),
            pl.BlockSpec((B, 1), const),
            pl.BlockSpec((B, S), const),
            pl.BlockSpec((B, S), const),
            pl.BlockSpec((B, _SC_LANES), const),
            pl.BlockSpec((B, 1), const),
        ],
        out_specs=[
            pl.BlockSpec((B, 1), const),
            pl.BlockSpec((B, 1), const),
            pl.BlockSpec((B, S), const),
            pl.BlockSpec((B, S), const),
            pl.BlockSpec(memory_space=pltpu.SMEM),
        ],
        scratch_shapes=[
            pltpu.VMEM((B, _VB), jnp.float32),
            pltpu.VMEM((B, 128 + (V % 128 or 128)), jnp.float32),
            pltpu.VMEM((B, _SC_CH), jnp.float32),
            pltpu.SemaphoreType.DMA,
        ],
    )
    tok, lti, am, gt, gi = pl.pallas_call(
        functools.partial(_pass_b, B=B, V=V, S=S),
        grid_spec=grid_spec,
        out_shape=(
            jax.ShapeDtypeStruct((B, 1), jnp.int32),
            jax.ShapeDtypeStruct((B, 1), jnp.int32),
            jax.ShapeDtypeStruct((B, S), attention_mask.dtype),
            jax.ShapeDtypeStruct((B, S), generated_tokens.dtype),
            jax.ShapeDtypeStruct((1,), jnp.int32),
        ),
        compiler_params=pltpu.CompilerParams(
            dimension_semantics=("arbitrary",),
        ),
    )(bid.reshape(B), woff.reshape(B), generated_index, logits, bid, vmax,
      last_token_index, attention_mask, generated_tokens, sc_max, woff)
    return tok, lti, am, gt, gi


# speedup vs baseline: 1.5452x; 1.3258x over previous
import functools

import jax
import jax.numpy as jnp
from jax.experimental import pallas as pl
from jax.experimental.pallas import tpu as pltpu

_VB = 65536


def _pass_a(x_ref, bid_out, max_out, vmax_ref, vbid_ref, *, B, V, NB):
    i = pl.program_id(0)

    @pl.when(i == 0)
    def _init():
        vmax_ref[...] = jnp.full((B, 1), -jnp.inf, jnp.float32)
        vbid_ref[...] = jnp.zeros((B, 1), jnp.int32)

    def _update(bmax):
        better = bmax > vmax_ref[...]
        vbid_ref[...] = jnp.where(better, i, vbid_ref[...])
        vmax_ref[...] = jnp.where(better, bmax, vmax_ref[...])

    @pl.when(i < NB - 1)
    def _full():
        _update(jnp.max(x_ref[...].reshape(B, _VB), axis=1, keepdims=True))

    @pl.when(i == NB - 1)
    def _tail():
        rem = V - (NB - 1) * _VB
        lidx = jax.lax.broadcasted_iota(jnp.int32, (B, _VB), 1)
        x = jnp.where(lidx < rem, x_ref[...].reshape(B, _VB), -jnp.inf)
        _update(jnp.max(x, axis=1, keepdims=True))
        bid_out[...] = vbid_ref[...]
        max_out[...] = vmax_ref[...]


def _pass_b(bid_sref, gi_ref, x_any, bidv_ref, max_ref, lti_ref, am_ref, gt_ref,
            tok_out, lti_out, am_out, gt_out, gi_out, xbuf, tbuf, sem, *, B, V, S):
    amax = ((V - _VB) // 128) * 128
    tw = 128 + (V % 128 or 128)
    toff = V - tw
    copies = []
    for b in range(B):
        off = pl.multiple_of(jnp.minimum(bid_sref[b] * _VB, amax), 128)
        copies.append(pltpu.make_async_copy(
            x_any.at[b, 0, pl.ds(off, _VB)], xbuf.at[b], sem))
        copies.append(pltpu.make_async_copy(
            x_any.at[b, 0, pl.ds(toff, tw)], tbuf.at[b], sem))
    for c in copies:
        c.start()
    for c in copies:
        c.wait()

    big = jnp.int32(2**31 - 1)
    x = xbuf[...]
    base = jnp.minimum(bidv_ref[...] * _VB, amax)
    lidx = jax.lax.broadcasted_iota(jnp.int32, (B, _VB), 1)
    cand = jnp.where(x == max_ref[...], lidx + base, big)
    m1 = jnp.min(cand, axis=1, keepdims=True)
    t = tbuf[...]
    tidx = jax.lax.broadcasted_iota(jnp.int32, (B, tw), 1) + toff
    cand2 = jnp.where(t == max_ref[...], tidx, big)
    m2 = jnp.min(cand2, axis=1, keepdims=True)
    tokens = jnp.minimum(m1, m2)
    tok_out[...] = tokens
    lti = jnp.minimum(lti_ref[...] + 1, S - 1)
    lti_out[...] = lti
    scol = jax.lax.broadcasted_iota(jnp.int32, (B, S), 1)
    am_out[...] = jnp.where(scol == lti, 1, am_ref[...])
    gi = gi_ref[0]
    gt_out[...] = jnp.where(scol == gi, tokens, gt_ref[...])
    gi_out[0] = jnp.minimum(gi + 1, S - 1)


def kernel(logits, last_token_index, attention_mask, generated_tokens, generated_index):
    B, _, V = logits.shape
    S = generated_tokens.shape[1]
    NB = pl.cdiv(V, _VB)

    bid, vmax = pl.pallas_call(
        functools.partial(_pass_a, B=B, V=V, NB=NB),
        grid=(NB,),
        in_specs=[pl.BlockSpec((B, 1, _VB), lambda i: (0, 0, i))],
        out_specs=[
            pl.BlockSpec((B, 1), lambda i: (0, 0)),
            pl.BlockSpec((B, 1), lambda i: (0, 0)),
        ],
        out_shape=(
            jax.ShapeDtypeStruct((B, 1), jnp.int32),
            jax.ShapeDtypeStruct((B, 1), jnp.float32),
        ),
        scratch_shapes=[
            pltpu.VMEM((B, 1), jnp.float32),
            pltpu.VMEM((B, 1), jnp.int32),
        ],
        compiler_params=pltpu.CompilerParams(
            dimension_semantics=("arbitrary",),
        ),
    )(logits)

    const = lambda i, bid_ref, gi_ref: (0, 0)
    grid_spec = pltpu.PrefetchScalarGridSpec(
        num_scalar_prefetch=2,
        grid=(1,),
        in_specs=[
            pl.BlockSpec(memory_space=pl.ANY),
            pl.BlockSpec((B, 1), const),
            pl.BlockSpec((B, 1), const),
            pl.BlockSpec((B, 1), const),
            pl.BlockSpec((B, S), const),
            pl.BlockSpec((B, S), const),
        ],
        out_specs=[
            pl.BlockSpec((B, 1), const),
            pl.BlockSpec((B, 1), const),
            pl.BlockSpec((B, S), const),
            pl.BlockSpec((B, S), const),
            pl.BlockSpec(memory_space=pltpu.SMEM),
        ],
        scratch_shapes=[
            pltpu.VMEM((B, _VB), jnp.float32),
            pltpu.VMEM((B, 128 + (V % 128 or 128)), jnp.float32),
            pltpu.SemaphoreType.DMA,
        ],
    )
    tok, lti, am, gt, gi = pl.pallas_call(
        functools.partial(_pass_b, B=B, V=V, S=S),
        grid_spec=grid_spec,
        out_shape=(
            jax.ShapeDtypeStruct((B, 1), jnp.int32),
            jax.ShapeDtypeStruct((B, 1), jnp.int32),
            jax.ShapeDtypeStruct((B, S), attention_mask.dtype),
            jax.ShapeDtypeStruct((B, S), generated_tokens.dtype),
            jax.ShapeDtypeStruct((1,), jnp.int32),
        ),
        compiler_params=pltpu.CompilerParams(
            dimension_semantics=("arbitrary",),
        ),
    )(bid.reshape(B), generated_index, logits, bid, vmax, last_token_index,
      attention_mask, generated_tokens)
    return tok, lti, am, gt, gi
